# baseline (speedup 1.0000x reference)
import functools

import jax
import jax.numpy as jnp
from jax import lax
from jax.experimental import pallas as pl
from jax.experimental.pallas import tpu as pltpu
from jax.experimental.pallas import tpu_sc as plsc

S = 2048
D_MODEL = 1024
D_FF = 4096
N_EXPERT = 8
TOP_K = 2
ROW_TILE = 256
BLK = 256
M = S * TOP_K + N_EXPERT * ROW_TILE
NT = M // ROW_TILE


def _gelu_new(x):
    return 0.5 * x * (1.0 + jnp.tanh(jnp.sqrt(2.0 / jnp.pi) * (x + 0.044715 * jnp.power(x, 3.0))))


def _router_body(x_ref, wg_ref, logits_ref, posk_ref, wpw_ref, te_ref,
                 oh0_s, oh1_s, r0_s, r1_s):
    x = x_ref[...]
    logits = lax.dot_general(x, wg_ref[...], (((1,), (0,)), ((), ())),
                             preferred_element_type=jnp.float32)
    logits_ref[...] = logits
    m = jnp.max(logits, axis=1, keepdims=True)
    ex = jnp.exp(logits - m)
    probs = ex / jnp.sum(ex, axis=1, keepdims=True)
    iota8 = lax.broadcasted_iota(jnp.int32, (S, N_EXPERT), 1)
    m1 = jnp.max(probs, axis=1, keepdims=True)
    e1 = jnp.min(jnp.where(probs == m1, iota8, N_EXPERT), axis=1, keepdims=True)
    masked = jnp.where(iota8 == e1, -jnp.inf, probs)
    m2 = jnp.max(masked, axis=1, keepdims=True)
    e2 = jnp.min(jnp.where(masked == m2, iota8, N_EXPERT), axis=1, keepdims=True)
    denom = m1 + m2
    w1 = m1 / denom
    w2 = m2 / denom

    oh0_s[...] = (iota8 == e1).astype(jnp.float32)
    oh1_s[...] = (iota8 == e2).astype(jnp.float32)

    rb = (lax.broadcasted_iota(jnp.int32, (BLK, BLK), 0)
          > lax.broadcasted_iota(jnp.int32, (BLK, BLK), 1)).astype(jnp.float32)

    def step(i, carry):
        c0, c1 = carry
        b0 = oh0_s[pl.ds(i * BLK, BLK), :]
        b1 = oh1_s[pl.ds(i * BLK, BLK), :]
        r0_s[pl.ds(i * BLK, BLK), :] = lax.dot_general(
            rb, b0, (((1,), (0,)), ((), ())), preferred_element_type=jnp.float32) + c0
        r1_s[pl.ds(i * BLK, BLK), :] = lax.dot_general(
            rb, b1, (((1,), (0,)), ((), ())), preferred_element_type=jnp.float32) + c1
        return (c0 + jnp.sum(b0, axis=0, keepdims=True),
                c1 + jnp.sum(b1, axis=0, keepdims=True))

    c0, c1 = lax.fori_loop(0, S // BLK, step,
                           (jnp.zeros((1, N_EXPERT), jnp.float32),
                            jnp.zeros((1, N_EXPERT), jnp.float32)))
    counts = c0 + c1
    pc = jnp.ceil(counts / ROW_TILE) * ROW_TILE
    tri8 = (lax.broadcasted_iota(jnp.int32, (N_EXPERT, N_EXPERT), 0)
            < lax.broadcasted_iota(jnp.int32, (N_EXPERT, N_EXPERT), 1)).astype(jnp.float32)
    po = lax.dot_general(pc, tri8, (((1,), (0,)), ((), ())),
                         preferred_element_type=jnp.float32)

    oh0 = oh0_s[...]
    oh1 = oh1_s[...]

    def sel(mat, oh):
        return jnp.sum(mat * oh, axis=1, keepdims=True)

    pos0 = sel(po, oh0) + sel(r0_s[...], oh0)
    pos1 = sel(po, oh1) + sel(c0, oh1) + sel(r1_s[...], oh1)
    posk = (jnp.where(iota8 == 0, pos0, 0.0)
            + jnp.where(iota8 == 1, pos1, 0.0))
    posk_ref[...] = jnp.transpose(posk).astype(jnp.int32)
    wpw_ref[pl.ds(0, S), :] = jnp.broadcast_to(w1, (S, 128))
    wpw_ref[pl.ds(S, S), :] = jnp.broadcast_to(w2, (S, 128))

    iota128 = lax.broadcasted_iota(jnp.int32, (1, 128), 1)
    ti = iota128.astype(jnp.float32) * ROW_TILE
    iota_e = lax.broadcasted_iota(jnp.int32, (1, N_EXPERT), 1)
    te = jnp.zeros((1, 128), jnp.float32)
    last_e = jnp.float32(0.0)
    for e in range(N_EXPERT):
        oh_e = (iota_e == e).astype(jnp.float32)
        po_e = jnp.sum(po * oh_e)
        pc_e = jnp.sum(pc * oh_e)
        te += (ti >= po_e).astype(jnp.float32)
        last_e = jnp.maximum(last_e, jnp.where(pc_e > 0, jnp.float32(e), 0.0))
    te = jnp.minimum(te - 1.0, last_e)
    te = jnp.maximum(te, 0.0)
    n_used = jnp.sum(pc) / ROW_TILE
    te = jnp.where(iota128 == 127, n_used, te)
    te_ref[...] = te.astype(jnp.int32)


@functools.lru_cache(maxsize=None)
def _get_dispatch():
    mesh = plsc.VectorSubcoreMesh(core_axis_name="c", subcore_axis_name="s")

    @functools.partial(
        pl.kernel, mesh=mesh,
        out_type=(
            jax.ShapeDtypeStruct((M, D_MODEL // 2), jnp.int32),
            jax.ShapeDtypeStruct((M, 128), jnp.float32),
        ),
        scratch_types=[
            pltpu.VMEM((64,), jnp.int32),
            pltpu.VMEM((64,), jnp.int32),
            pltpu.VMEM((64, D_MODEL // 2), jnp.int32),
            pltpu.VMEM((64, D_MODEL // 2), jnp.int32),
            pltpu.VMEM((64, 128), jnp.float32),
            pltpu.VMEM((64, 128), jnp.float32),
            pltpu.SemaphoreType.DMA,
        ],
    )
    def _dispatch(hs_hbm, posk_hbm, wpw_hbm, xd_hbm, ws_hbm,
                  idx0_v, idx1_v, rows0_v, rows1_v, w0_v, w1_v, sem):
        wid = lax.axis_index("s") * 2 + lax.axis_index("c")
        k = wid // 16
        t0 = (wid % 16) * 128
        p0 = k * S + t0
        pltpu.sync_copy(posk_hbm.at[pl.ds(p0, 64)], idx0_v)
        pltpu.sync_copy(posk_hbm.at[pl.ds(p0 + 64, 64)], idx1_v)
        pltpu.sync_copy(hs_hbm.at[pl.ds(t0, 64)], rows0_v)
        pltpu.sync_copy(wpw_hbm.at[pl.ds(p0, 64)], w0_v)
        c0 = pltpu.async_copy(rows0_v, xd_hbm.at[idx0_v], sem)
        c1 = pltpu.async_copy(w0_v, ws_hbm.at[idx0_v], sem)
        pltpu.sync_copy(hs_hbm.at[pl.ds(t0 + 64, 64)], rows1_v)
        pltpu.sync_copy(wpw_hbm.at[pl.ds(p0 + 64, 64)], w1_v)
        c2 = pltpu.async_copy(rows1_v, xd_hbm.at[idx1_v], sem)
        c3 = pltpu.async_copy(w1_v, ws_hbm.at[idx1_v], sem)
        c0.wait()
        c1.wait()
        c2.wait()
        c3.wait()

    return _dispatch


def _ffn_body(te_ref, x_ref, wfc_ref, bfc_ref, wproj_ref, bproj_ref, ws_ref, y_ref):
    i = pl.program_id(0)

    @pl.when(i < te_ref[127])
    def _():
        x = x_ref[...]
        h = lax.dot_general(x, wfc_ref[0], (((1,), (0,)), ((), ())),
                            preferred_element_type=jnp.float32)
        h = _gelu_new(h + bfc_ref[0])
        y = lax.dot_general(h.astype(jnp.bfloat16), wproj_ref[0], (((1,), (0,)), ((), ())),
                            preferred_element_type=jnp.float32)
        w_col = ws_ref[...][:, 0:1]
        y_ref[...] = (y + bproj_ref[0]) * w_col


@functools.lru_cache(maxsize=None)
def _get_combine():
    mesh = plsc.VectorSubcoreMesh(core_axis_name="c", subcore_axis_name="s")

    @functools.partial(
        pl.kernel, mesh=mesh,
        out_type=jax.ShapeDtypeStruct((S, D_MODEL), jnp.float32),
        scratch_types=[
            pltpu.VMEM((16,), jnp.int32),
            pltpu.VMEM((16,), jnp.int32),
            pltpu.VMEM((16,), jnp.int32),
            pltpu.VMEM((16,), jnp.int32),
            pltpu.VMEM((16, D_MODEL), jnp.float32),
            pltpu.VMEM((16, D_MODEL), jnp.float32),
            pltpu.VMEM((16, D_MODEL), jnp.float32),
            pltpu.VMEM((16, D_MODEL), jnp.float32),
            pltpu.VMEM((16, D_MODEL), jnp.float32),
            pltpu.VMEM((16, D_MODEL), jnp.float32),
            pltpu.SemaphoreType.DMA,
        ],
    )
    def _combine(y_hbm, posk_hbm, out_hbm,
                 idx0a_v, idx1a_v, idx0b_v, idx1b_v,
                 y0a_v, y1a_v, y0b_v, y1b_v, oa_v, ob_v, sem):
        wid = lax.axis_index("s") * 2 + lax.axis_index("c")
        t0 = wid * 64
        CH = 16
        bufs = ((idx0a_v, idx1a_v, y0a_v, y1a_v, oa_v),
                (idx0b_v, idx1b_v, y0b_v, y1b_v, ob_v))

        def fire(c, bi):
            i0, i1, yy0, yy1, _ = bufs[bi]
            tc = t0 + c * CH
            pltpu.sync_copy(posk_hbm.at[pl.ds(tc, CH)], i0)
            pltpu.sync_copy(posk_hbm.at[pl.ds(S + tc, CH)], i1)
            return (pltpu.async_copy(y_hbm.at[i0], yy0, sem),
                    pltpu.async_copy(y_hbm.at[i1], yy1, sem))

        pend = fire(0, 0)
        for c in range(4):
            bi = c % 2
            nxt = fire(c + 1, 1 - bi) if c < 3 else None
            pend[0].wait()
            pend[1].wait()
            _, _, yy0, yy1, oo = bufs[bi]

            def tok(i, carry, yy0=yy0, yy1=yy1, oo=oo):
                for cc in range(D_MODEL // 16):
                    oo[i, pl.ds(cc * 16, 16)] = (yy0[i, pl.ds(cc * 16, 16)]
                                                 + yy1[i, pl.ds(cc * 16, 16)])
                return carry

            lax.fori_loop(0, CH, tok, 0)
            pltpu.sync_copy(oo, out_hbm.at[pl.ds(t0 + c * CH, CH)])
            pend = nxt

    return _combine


def kernel(hidden_states, W_g, c_fc_w, c_fc_b, c_proj_w, c_proj_b):
    b, s, d = hidden_states.shape
    hs = hidden_states.reshape(s, d)

    logits, posk, wpw, te = pl.pallas_call(
        _router_body,
        out_shape=(
            jax.ShapeDtypeStruct((S, N_EXPERT), jnp.float32),
            jax.ShapeDtypeStruct((N_EXPERT, S), jnp.int32),
            jax.ShapeDtypeStruct((TOP_K * S, 128), jnp.float32),
            jax.ShapeDtypeStruct((1, 128), jnp.int32),
        ),
        scratch_shapes=[pltpu.VMEM((S, N_EXPERT), jnp.float32)] * 4,
    )(hs, W_g)

    posk_flat = posk[:TOP_K].reshape(TOP_K * S)
    hs_bf = hs.astype(jnp.bfloat16)
    hs_i32 = lax.bitcast_convert_type(
        hs_bf.reshape(S, D_MODEL // 2, 2), jnp.int32)
    x_disp_i32, w_sorted = _get_dispatch()(hs_i32, posk_flat, wpw)
    x_disp = lax.bitcast_convert_type(x_disp_i32, jnp.bfloat16).reshape(M, D_MODEL)

    wfc = c_fc_w.astype(jnp.bfloat16)
    wproj = c_proj_w.astype(jnp.bfloat16)

    y = pl.pallas_call(
        _ffn_body,
        grid_spec=pltpu.PrefetchScalarGridSpec(
            num_scalar_prefetch=1,
            grid=(NT,),
            in_specs=[
                pl.BlockSpec((ROW_TILE, D_MODEL), lambda i, te: (i, 0)),
                pl.BlockSpec((1, D_MODEL, D_FF), lambda i, te: (te[i], 0, 0)),
                pl.BlockSpec((1, 1, D_FF), lambda i, te: (te[i], 0, 0)),
                pl.BlockSpec((1, D_FF, D_MODEL), lambda i, te: (te[i], 0, 0)),
                pl.BlockSpec((1, 1, D_MODEL), lambda i, te: (te[i], 0, 0)),
                pl.BlockSpec((ROW_TILE, 128), lambda i, te: (i, 0)),
            ],
            out_specs=pl.BlockSpec((ROW_TILE, D_MODEL), lambda i, te: (i, 0)),
        ),
        out_shape=jax.ShapeDtypeStruct((M, D_MODEL), jnp.float32),
    )(te.reshape(128), x_disp, wfc, c_fc_b.reshape(N_EXPERT, 1, D_FF),
      wproj, c_proj_b.reshape(N_EXPERT, 1, D_MODEL), w_sorted)

    final = _get_combine()(y, posk_flat)
    return (final.reshape(b, s, d), logits)


# speedup vs baseline: 1.5765x; 1.5765x over previous
import functools

import jax
import jax.numpy as jnp
from jax import lax
from jax.experimental import pallas as pl
from jax.experimental.pallas import tpu as pltpu
from jax.experimental.pallas import tpu_sc as plsc

S = 2048
D_MODEL = 1024
D_FF = 4096
N_EXPERT = 8
TOP_K = 2
ROW_TILE = 256
BLK = 256
M = S * TOP_K + N_EXPERT * ROW_TILE
NT = M // ROW_TILE


def _gelu_new(x):
    return 0.5 * x * (1.0 + jnp.tanh(jnp.sqrt(2.0 / jnp.pi) * (x + 0.044715 * jnp.power(x, 3.0))))


def _router_body(x_ref, wg_ref, logits_ref, posk_ref, wpw_ref, te_ref,
                 oh0_s, oh1_s, r0_s, r1_s):
    x = x_ref[...]
    logits = lax.dot_general(x, wg_ref[...], (((1,), (0,)), ((), ())),
                             preferred_element_type=jnp.float32)
    logits_ref[...] = logits
    m = jnp.max(logits, axis=1, keepdims=True)
    ex = jnp.exp(logits - m)
    probs = ex / jnp.sum(ex, axis=1, keepdims=True)
    iota8 = lax.broadcasted_iota(jnp.int32, (S, N_EXPERT), 1)
    m1 = jnp.max(probs, axis=1, keepdims=True)
    e1 = jnp.min(jnp.where(probs == m1, iota8, N_EXPERT), axis=1, keepdims=True)
    masked = jnp.where(iota8 == e1, -jnp.inf, probs)
    m2 = jnp.max(masked, axis=1, keepdims=True)
    e2 = jnp.min(jnp.where(masked == m2, iota8, N_EXPERT), axis=1, keepdims=True)
    denom = m1 + m2
    w1 = m1 / denom
    w2 = m2 / denom

    oh0_s[...] = (iota8 == e1).astype(jnp.float32)
    oh1_s[...] = (iota8 == e2).astype(jnp.float32)

    rb = (lax.broadcasted_iota(jnp.int32, (BLK, BLK), 0)
          > lax.broadcasted_iota(jnp.int32, (BLK, BLK), 1)).astype(jnp.float32)

    def step(i, carry):
        c0, c1 = carry
        b0 = oh0_s[pl.ds(i * BLK, BLK), :]
        b1 = oh1_s[pl.ds(i * BLK, BLK), :]
        r0_s[pl.ds(i * BLK, BLK), :] = lax.dot_general(
            rb, b0, (((1,), (0,)), ((), ())), preferred_element_type=jnp.float32) + c0
        r1_s[pl.ds(i * BLK, BLK), :] = lax.dot_general(
            rb, b1, (((1,), (0,)), ((), ())), preferred_element_type=jnp.float32) + c1
        return (c0 + jnp.sum(b0, axis=0, keepdims=True),
                c1 + jnp.sum(b1, axis=0, keepdims=True))

    c0, c1 = lax.fori_loop(0, S // BLK, step,
                           (jnp.zeros((1, N_EXPERT), jnp.float32),
                            jnp.zeros((1, N_EXPERT), jnp.float32)))
    counts = c0 + c1
    pc = jnp.ceil(counts / ROW_TILE) * ROW_TILE
    tri8 = (lax.broadcasted_iota(jnp.int32, (N_EXPERT, N_EXPERT), 0)
            < lax.broadcasted_iota(jnp.int32, (N_EXPERT, N_EXPERT), 1)).astype(jnp.float32)
    po = lax.dot_general(pc, tri8, (((1,), (0,)), ((), ())),
                         preferred_element_type=jnp.float32)

    oh0 = oh0_s[...]
    oh1 = oh1_s[...]

    def sel(mat, oh):
        return jnp.sum(mat * oh, axis=1, keepdims=True)

    pos0 = sel(po, oh0) + sel(r0_s[...], oh0)
    pos1 = sel(po, oh1) + sel(c0, oh1) + sel(r1_s[...], oh1)
    posk = (jnp.where(iota8 == 0, pos0, 0.0)
            + jnp.where(iota8 == 1, pos1, 0.0))
    posk_ref[...] = jnp.transpose(posk).astype(jnp.int32)
    wpw_ref[pl.ds(0, S), :] = jnp.broadcast_to(w1, (S, 128))
    wpw_ref[pl.ds(S, S), :] = jnp.broadcast_to(w2, (S, 128))

    iota128 = lax.broadcasted_iota(jnp.int32, (1, 128), 1)
    ti = iota128.astype(jnp.float32) * ROW_TILE
    iota_e = lax.broadcasted_iota(jnp.int32, (1, N_EXPERT), 1)
    te = jnp.zeros((1, 128), jnp.float32)
    last_e = jnp.float32(0.0)
    for e in range(N_EXPERT):
        oh_e = (iota_e == e).astype(jnp.float32)
        po_e = jnp.sum(po * oh_e)
        pc_e = jnp.sum(pc * oh_e)
        te += (ti >= po_e).astype(jnp.float32)
        last_e = jnp.maximum(last_e, jnp.where(pc_e > 0, jnp.float32(e), 0.0))
    te = jnp.minimum(te - 1.0, last_e)
    te = jnp.maximum(te, 0.0)
    n_used = jnp.sum(pc) / ROW_TILE
    te = jnp.where(iota128 == 127, n_used, te)
    te_ref[...] = te.astype(jnp.int32)


@functools.lru_cache(maxsize=None)
def _get_dispatch():
    mesh = plsc.VectorSubcoreMesh(core_axis_name="c", subcore_axis_name="s")

    @functools.partial(
        pl.kernel, mesh=mesh,
        out_type=(
            jax.ShapeDtypeStruct((M, D_MODEL), jnp.float32),
            jax.ShapeDtypeStruct((M, 128), jnp.float32),
        ),
        scratch_types=[
            pltpu.VMEM((32,), jnp.int32),
            pltpu.VMEM((32,), jnp.int32),
            pltpu.VMEM((32, D_MODEL), jnp.float32),
            pltpu.VMEM((32, D_MODEL), jnp.float32),
            pltpu.VMEM((32, 128), jnp.float32),
            pltpu.VMEM((32, 128), jnp.float32),
            pltpu.SemaphoreType.DMA,
            pltpu.SemaphoreType.DMA,
        ],
    )
    def _dispatch(hs_hbm, posk_hbm, wpw_hbm, xd_hbm, ws_hbm,
                  idx0_v, idx1_v, rows0_v, rows1_v, w0_v, w1_v, sem_in, sem_out):
        wid = lax.axis_index("s") * 2 + lax.axis_index("c")
        k = wid // 16
        t0 = (wid % 16) * 128
        CH = 32
        bufs = ((idx0_v, rows0_v, w0_v), (idx1_v, rows1_v, w1_v))

        def stage(c, bi):
            i_v, r_v, w_v = bufs[bi]
            tc = t0 + c * CH
            pltpu.sync_copy(posk_hbm.at[pl.ds(k * S + tc, CH)], i_v)
            return (pltpu.async_copy(hs_hbm.at[pl.ds(tc, CH)], r_v, sem_in),
                    pltpu.async_copy(wpw_hbm.at[pl.ds(k * S + tc, CH)], w_v, sem_in))

        def scatter(bi):
            i_v, r_v, w_v = bufs[bi]
            return (pltpu.async_copy(r_v, xd_hbm.at[i_v], sem_out),
                    pltpu.async_copy(w_v, ws_hbm.at[i_v], sem_out))

        pend_sc = None
        st = stage(0, 0)
        for c in range(4):
            bi = c % 2
            if pend_sc is not None:
                pend_sc[0].wait()
                pend_sc[1].wait()
            stn = stage(c + 1, 1 - bi) if c < 3 else None
            st[0].wait()
            st[1].wait()
            pend_sc = scatter(bi)
            st = stn
        pend_sc[0].wait()
        pend_sc[1].wait()

    return _dispatch


def _ffn_body(te_ref, x_ref, wfc_ref, bfc_ref, wproj_ref, bproj_ref, ws_ref, y_ref):
    i = pl.program_id(0)

    @pl.when(i < te_ref[127])
    def _():
        x = x_ref[...].astype(jnp.bfloat16)
        h = lax.dot_general(x, wfc_ref[0], (((1,), (0,)), ((), ())),
                            preferred_element_type=jnp.float32)
        h = _gelu_new(h + bfc_ref[0])
        y = lax.dot_general(h.astype(jnp.bfloat16), wproj_ref[0], (((1,), (0,)), ((), ())),
                            preferred_element_type=jnp.float32)
        w_col = ws_ref[...][:, 0:1]
        y_ref[...] = (y + bproj_ref[0]) * w_col


@functools.lru_cache(maxsize=None)
def _get_combine():
    mesh = plsc.VectorSubcoreMesh(core_axis_name="c", subcore_axis_name="s")

    @functools.partial(
        pl.kernel, mesh=mesh,
        out_type=jax.ShapeDtypeStruct((S, D_MODEL), jnp.float32),
        scratch_types=[
            pltpu.VMEM((16,), jnp.int32),
            pltpu.VMEM((16,), jnp.int32),
            pltpu.VMEM((16,), jnp.int32),
            pltpu.VMEM((16,), jnp.int32),
            pltpu.VMEM((16, D_MODEL), jnp.float32),
            pltpu.VMEM((16, D_MODEL), jnp.float32),
            pltpu.VMEM((16, D_MODEL), jnp.float32),
            pltpu.VMEM((16, D_MODEL), jnp.float32),
            pltpu.VMEM((16, D_MODEL), jnp.float32),
            pltpu.VMEM((16, D_MODEL), jnp.float32),
            pltpu.SemaphoreType.DMA,
        ],
    )
    def _combine(y_hbm, posk_hbm, out_hbm,
                 idx0a_v, idx1a_v, idx0b_v, idx1b_v,
                 y0a_v, y1a_v, y0b_v, y1b_v, oa_v, ob_v, sem):
        wid = lax.axis_index("s") * 2 + lax.axis_index("c")
        t0 = wid * 64
        CH = 16
        bufs = ((idx0a_v, idx1a_v, y0a_v, y1a_v, oa_v),
                (idx0b_v, idx1b_v, y0b_v, y1b_v, ob_v))

        def fire(c, bi):
            i0, i1, yy0, yy1, _ = bufs[bi]
            tc = t0 + c * CH
            pltpu.sync_copy(posk_hbm.at[pl.ds(tc, CH)], i0)
            pltpu.sync_copy(posk_hbm.at[pl.ds(S + tc, CH)], i1)
            return (pltpu.async_copy(y_hbm.at[i0], yy0, sem),
                    pltpu.async_copy(y_hbm.at[i1], yy1, sem))

        pend = fire(0, 0)
        for c in range(4):
            bi = c % 2
            nxt = fire(c + 1, 1 - bi) if c < 3 else None
            pend[0].wait()
            pend[1].wait()
            _, _, yy0, yy1, oo = bufs[bi]

            def tok(i, carry, yy0=yy0, yy1=yy1, oo=oo):
                for cc in range(D_MODEL // 16):
                    oo[i, pl.ds(cc * 16, 16)] = (yy0[i, pl.ds(cc * 16, 16)]
                                                 + yy1[i, pl.ds(cc * 16, 16)])
                return carry

            lax.fori_loop(0, CH, tok, 0)
            pltpu.sync_copy(oo, out_hbm.at[pl.ds(t0 + c * CH, CH)])
            pend = nxt

    return _combine


def kernel(hidden_states, W_g, c_fc_w, c_fc_b, c_proj_w, c_proj_b):
    b, s, d = hidden_states.shape
    hs = hidden_states.reshape(s, d)

    logits, posk, wpw, te = pl.pallas_call(
        _router_body,
        out_shape=(
            jax.ShapeDtypeStruct((S, N_EXPERT), jnp.float32),
            jax.ShapeDtypeStruct((N_EXPERT, S), jnp.int32),
            jax.ShapeDtypeStruct((TOP_K * S, 128), jnp.float32),
            jax.ShapeDtypeStruct((1, 128), jnp.int32),
        ),
        scratch_shapes=[pltpu.VMEM((S, N_EXPERT), jnp.float32)] * 4,
    )(hs, W_g)

    posk_flat = posk[:TOP_K].reshape(TOP_K * S)
    x_disp, w_sorted = _get_dispatch()(hs, posk_flat, wpw)

    wfc = c_fc_w.astype(jnp.bfloat16)
    wproj = c_proj_w.astype(jnp.bfloat16)

    y = pl.pallas_call(
        _ffn_body,
        grid_spec=pltpu.PrefetchScalarGridSpec(
            num_scalar_prefetch=1,
            grid=(NT,),
            in_specs=[
                pl.BlockSpec((ROW_TILE, D_MODEL), lambda i, te: (i, 0)),
                pl.BlockSpec((1, D_MODEL, D_FF), lambda i, te: (te[i], 0, 0)),
                pl.BlockSpec((1, 1, D_FF), lambda i, te: (te[i], 0, 0)),
                pl.BlockSpec((1, D_FF, D_MODEL), lambda i, te: (te[i], 0, 0)),
                pl.BlockSpec((1, 1, D_MODEL), lambda i, te: (te[i], 0, 0)),
                pl.BlockSpec((ROW_TILE, 128), lambda i, te: (i, 0)),
            ],
            out_specs=pl.BlockSpec((ROW_TILE, D_MODEL), lambda i, te: (i, 0)),
        ),
        out_shape=jax.ShapeDtypeStruct((M, D_MODEL), jnp.float32),
    )(te.reshape(128), x_disp, wfc, c_fc_b.reshape(N_EXPERT, 1, D_FF),
      wproj, c_proj_b.reshape(N_EXPERT, 1, D_MODEL), w_sorted)

    final = _get_combine()(y, posk_flat)
    return (final.reshape(b, s, d), logits)


# speedup vs baseline: 1.5894x; 1.0082x over previous
import functools

import jax
import jax.numpy as jnp
from jax import lax
from jax.experimental import pallas as pl
from jax.experimental.pallas import tpu as pltpu
from jax.experimental.pallas import tpu_sc as plsc

S = 2048
D_MODEL = 1024
D_FF = 4096
N_EXPERT = 8
TOP_K = 2
ROW_TILE = 256
BLK = 256
M = S * TOP_K + N_EXPERT * ROW_TILE
NT = M // ROW_TILE


def _gelu_new(x):
    return 0.5 * x * (1.0 + jnp.tanh(jnp.sqrt(2.0 / jnp.pi) * (x + 0.044715 * jnp.power(x, 3.0))))


def _router_body(x_ref, wg_ref, logits_ref, posk_ref, wpw_ref, te_ref,
                 oh0_s, oh1_s, r0_s, r1_s):
    x = x_ref[...]
    logits = lax.dot_general(x, wg_ref[...], (((1,), (0,)), ((), ())),
                             preferred_element_type=jnp.float32)
    logits_ref[...] = logits
    m = jnp.max(logits, axis=1, keepdims=True)
    ex = jnp.exp(logits - m)
    probs = ex / jnp.sum(ex, axis=1, keepdims=True)
    iota8 = lax.broadcasted_iota(jnp.int32, (S, N_EXPERT), 1)
    m1 = jnp.max(probs, axis=1, keepdims=True)
    e1 = jnp.min(jnp.where(probs == m1, iota8, N_EXPERT), axis=1, keepdims=True)
    masked = jnp.where(iota8 == e1, -jnp.inf, probs)
    m2 = jnp.max(masked, axis=1, keepdims=True)
    e2 = jnp.min(jnp.where(masked == m2, iota8, N_EXPERT), axis=1, keepdims=True)
    denom = m1 + m2
    w1 = m1 / denom
    w2 = m2 / denom

    oh0_s[...] = (iota8 == e1).astype(jnp.float32)
    oh1_s[...] = (iota8 == e2).astype(jnp.float32)

    rb = (lax.broadcasted_iota(jnp.int32, (BLK, BLK), 0)
          > lax.broadcasted_iota(jnp.int32, (BLK, BLK), 1)).astype(jnp.float32)

    def step(i, carry):
        c0, c1 = carry
        b0 = oh0_s[pl.ds(i * BLK, BLK), :]
        b1 = oh1_s[pl.ds(i * BLK, BLK), :]
        r0_s[pl.ds(i * BLK, BLK), :] = lax.dot_general(
            rb, b0, (((1,), (0,)), ((), ())), preferred_element_type=jnp.float32) + c0
        r1_s[pl.ds(i * BLK, BLK), :] = lax.dot_general(
            rb, b1, (((1,), (0,)), ((), ())), preferred_element_type=jnp.float32) + c1
        return (c0 + jnp.sum(b0, axis=0, keepdims=True),
                c1 + jnp.sum(b1, axis=0, keepdims=True))

    c0, c1 = lax.fori_loop(0, S // BLK, step,
                           (jnp.zeros((1, N_EXPERT), jnp.float32),
                            jnp.zeros((1, N_EXPERT), jnp.float32)))
    counts = c0 + c1
    pc = jnp.ceil(counts / ROW_TILE) * ROW_TILE
    tri8 = (lax.broadcasted_iota(jnp.int32, (N_EXPERT, N_EXPERT), 0)
            < lax.broadcasted_iota(jnp.int32, (N_EXPERT, N_EXPERT), 1)).astype(jnp.float32)
    po = lax.dot_general(pc, tri8, (((1,), (0,)), ((), ())),
                         preferred_element_type=jnp.float32)

    oh0 = oh0_s[...]
    oh1 = oh1_s[...]

    def sel(mat, oh):
        return jnp.sum(mat * oh, axis=1, keepdims=True)

    pos0 = sel(po, oh0) + sel(r0_s[...], oh0)
    pos1 = sel(po, oh1) + sel(c0, oh1) + sel(r1_s[...], oh1)
    posk = (jnp.where(iota8 == 0, pos0, 0.0)
            + jnp.where(iota8 == 1, pos1, 0.0))
    posk_ref[...] = jnp.transpose(posk).astype(jnp.int32)
    wpw_ref[pl.ds(0, S), :] = jnp.broadcast_to(w1, (S, 128))
    wpw_ref[pl.ds(S, S), :] = jnp.broadcast_to(w2, (S, 128))

    iota128 = lax.broadcasted_iota(jnp.int32, (1, 128), 1)
    ti = iota128.astype(jnp.float32) * ROW_TILE
    iota_e = lax.broadcasted_iota(jnp.int32, (1, N_EXPERT), 1)
    te = jnp.zeros((1, 128), jnp.float32)
    last_e = jnp.float32(0.0)
    for e in range(N_EXPERT):
        oh_e = (iota_e == e).astype(jnp.float32)
        po_e = jnp.sum(po * oh_e)
        pc_e = jnp.sum(pc * oh_e)
        te += (ti >= po_e).astype(jnp.float32)
        last_e = jnp.maximum(last_e, jnp.where(pc_e > 0, jnp.float32(e), 0.0))
    te = jnp.minimum(te - 1.0, last_e)
    te = jnp.maximum(te, 0.0)
    n_used = jnp.sum(pc) / ROW_TILE
    te = jnp.where(iota128 == 127, n_used, te)
    te_ref[...] = te.astype(jnp.int32)


@functools.lru_cache(maxsize=None)
def _get_dispatch():
    mesh = plsc.VectorSubcoreMesh(core_axis_name="c", subcore_axis_name="s")

    @functools.partial(
        pl.kernel, mesh=mesh,
        out_type=(
            jax.ShapeDtypeStruct((M, D_MODEL), jnp.float32),
            jax.ShapeDtypeStruct((M, 128), jnp.float32),
        ),
        scratch_types=[
            pltpu.VMEM((32,), jnp.int32),
            pltpu.VMEM((32,), jnp.int32),
            pltpu.VMEM((32, D_MODEL), jnp.float32),
            pltpu.VMEM((32, D_MODEL), jnp.float32),
            pltpu.VMEM((32, 128), jnp.float32),
            pltpu.VMEM((32, 128), jnp.float32),
            pltpu.SemaphoreType.DMA,
            pltpu.SemaphoreType.DMA,
        ],
    )
    def _dispatch(hs_hbm, posk_hbm, wpw_hbm, xd_hbm, ws_hbm,
                  idx0_v, idx1_v, rows0_v, rows1_v, w0_v, w1_v, sem_in, sem_out):
        wid = lax.axis_index("s") * 2 + lax.axis_index("c")
        k = wid // 16
        t0 = (wid % 16) * 128
        CH = 32
        bufs = ((idx0_v, rows0_v, w0_v), (idx1_v, rows1_v, w1_v))

        def stage(c, bi):
            i_v, r_v, w_v = bufs[bi]
            tc = t0 + c * CH
            pltpu.sync_copy(posk_hbm.at[pl.ds(k * S + tc, CH)], i_v)
            return (pltpu.async_copy(hs_hbm.at[pl.ds(tc, CH)], r_v, sem_in),
                    pltpu.async_copy(wpw_hbm.at[pl.ds(k * S + tc, CH)], w_v, sem_in))

        def scatter(bi):
            i_v, r_v, w_v = bufs[bi]
            return (pltpu.async_copy(r_v, xd_hbm.at[i_v], sem_out),
                    pltpu.async_copy(w_v, ws_hbm.at[i_v], sem_out))

        pend_sc = None
        st = stage(0, 0)
        for c in range(4):
            bi = c % 2
            if pend_sc is not None:
                pend_sc[0].wait()
                pend_sc[1].wait()
            stn = stage(c + 1, 1 - bi) if c < 3 else None
            st[0].wait()
            st[1].wait()
            pend_sc = scatter(bi)
            st = stn
        pend_sc[0].wait()
        pend_sc[1].wait()

    return _dispatch


def _cast_body(wfc_ref, wproj_ref, ofc_ref, oproj_ref):
    ofc_ref[...] = wfc_ref[...].astype(jnp.bfloat16)
    oproj_ref[...] = wproj_ref[...].astype(jnp.bfloat16)


def _cast_weights(c_fc_w, c_proj_w):
    return pl.pallas_call(
        _cast_body,
        grid=(N_EXPERT, 4),
        in_specs=[
            pl.BlockSpec((1, D_MODEL // 4, D_FF), lambda e, c: (e, c, 0)),
            pl.BlockSpec((1, D_FF // 4, D_MODEL), lambda e, c: (e, c, 0)),
        ],
        out_specs=[
            pl.BlockSpec((1, D_MODEL // 4, D_FF), lambda e, c: (e, c, 0)),
            pl.BlockSpec((1, D_FF // 4, D_MODEL), lambda e, c: (e, c, 0)),
        ],
        out_shape=(
            jax.ShapeDtypeStruct((N_EXPERT, D_MODEL, D_FF), jnp.bfloat16),
            jax.ShapeDtypeStruct((N_EXPERT, D_FF, D_MODEL), jnp.bfloat16),
        ),
    )(c_fc_w, c_proj_w)


def _ffn_body(te_ref, x_ref, wfc_ref, bfc_ref, wproj_ref, bproj_ref, ws_ref, y_ref):
    i = pl.program_id(0)

    @pl.when(i < te_ref[127])
    def _():
        x = x_ref[...].astype(jnp.bfloat16)
        h = lax.dot_general(x, wfc_ref[0], (((1,), (0,)), ((), ())),
                            preferred_element_type=jnp.float32)
        h = _gelu_new(h + bfc_ref[0])
        y = lax.dot_general(h.astype(jnp.bfloat16), wproj_ref[0], (((1,), (0,)), ((), ())),
                            preferred_element_type=jnp.float32)
        w_col = ws_ref[...][:, 0:1]
        y_ref[...] = (y + bproj_ref[0]) * w_col


@functools.lru_cache(maxsize=None)
def _get_combine():
    mesh = plsc.VectorSubcoreMesh(core_axis_name="c", subcore_axis_name="s")

    @functools.partial(
        pl.kernel, mesh=mesh,
        out_type=jax.ShapeDtypeStruct((S, D_MODEL), jnp.float32),
        scratch_types=[
            pltpu.VMEM((16,), jnp.int32),
            pltpu.VMEM((16,), jnp.int32),
            pltpu.VMEM((16,), jnp.int32),
            pltpu.VMEM((16,), jnp.int32),
            pltpu.VMEM((16, D_MODEL), jnp.float32),
            pltpu.VMEM((16, D_MODEL), jnp.float32),
            pltpu.VMEM((16, D_MODEL), jnp.float32),
            pltpu.VMEM((16, D_MODEL), jnp.float32),
            pltpu.VMEM((16, D_MODEL), jnp.float32),
            pltpu.VMEM((16, D_MODEL), jnp.float32),
            pltpu.SemaphoreType.DMA,
        ],
    )
    def _combine(y_hbm, posk_hbm, out_hbm,
                 idx0a_v, idx1a_v, idx0b_v, idx1b_v,
                 y0a_v, y1a_v, y0b_v, y1b_v, oa_v, ob_v, sem):
        wid = lax.axis_index("s") * 2 + lax.axis_index("c")
        t0 = wid * 64
        CH = 16
        bufs = ((idx0a_v, idx1a_v, y0a_v, y1a_v, oa_v),
                (idx0b_v, idx1b_v, y0b_v, y1b_v, ob_v))

        def fire(c, bi):
            i0, i1, yy0, yy1, _ = bufs[bi]
            tc = t0 + c * CH
            pltpu.sync_copy(posk_hbm.at[pl.ds(tc, CH)], i0)
            pltpu.sync_copy(posk_hbm.at[pl.ds(S + tc, CH)], i1)
            return (pltpu.async_copy(y_hbm.at[i0], yy0, sem),
                    pltpu.async_copy(y_hbm.at[i1], yy1, sem))

        pend = fire(0, 0)
        for c in range(4):
            bi = c % 2
            nxt = fire(c + 1, 1 - bi) if c < 3 else None
            pend[0].wait()
            pend[1].wait()
            _, _, yy0, yy1, oo = bufs[bi]

            def tok(i, carry, yy0=yy0, yy1=yy1, oo=oo):
                for cc in range(D_MODEL // 16):
                    oo[i, pl.ds(cc * 16, 16)] = (yy0[i, pl.ds(cc * 16, 16)]
                                                 + yy1[i, pl.ds(cc * 16, 16)])
                return carry

            lax.fori_loop(0, CH, tok, 0)
            pltpu.sync_copy(oo, out_hbm.at[pl.ds(t0 + c * CH, CH)])
            pend = nxt

    return _combine


def kernel(hidden_states, W_g, c_fc_w, c_fc_b, c_proj_w, c_proj_b):
    b, s, d = hidden_states.shape
    hs = hidden_states.reshape(s, d)

    logits, posk, wpw, te = pl.pallas_call(
        _router_body,
        out_shape=(
            jax.ShapeDtypeStruct((S, N_EXPERT), jnp.float32),
            jax.ShapeDtypeStruct((N_EXPERT, S), jnp.int32),
            jax.ShapeDtypeStruct((TOP_K * S, 128), jnp.float32),
            jax.ShapeDtypeStruct((1, 128), jnp.int32),
        ),
        scratch_shapes=[pltpu.VMEM((S, N_EXPERT), jnp.float32)] * 4,
    )(hs, W_g)

    posk_flat = posk[:TOP_K].reshape(TOP_K * S)
    x_disp, w_sorted = _get_dispatch()(hs, posk_flat, wpw)

    wfc, wproj = _cast_weights(c_fc_w, c_proj_w)

    y = pl.pallas_call(
        _ffn_body,
        grid_spec=pltpu.PrefetchScalarGridSpec(
            num_scalar_prefetch=1,
            grid=(NT,),
            in_specs=[
                pl.BlockSpec((ROW_TILE, D_MODEL), lambda i, te: (i, 0)),
                pl.BlockSpec((1, D_MODEL, D_FF), lambda i, te: (te[i], 0, 0)),
                pl.BlockSpec((1, 1, D_FF), lambda i, te: (te[i], 0, 0)),
                pl.BlockSpec((1, D_FF, D_MODEL), lambda i, te: (te[i], 0, 0)),
                pl.BlockSpec((1, 1, D_MODEL), lambda i, te: (te[i], 0, 0)),
                pl.BlockSpec((ROW_TILE, 128), lambda i, te: (i, 0)),
            ],
            out_specs=pl.BlockSpec((ROW_TILE, D_MODEL), lambda i, te: (i, 0)),
        ),
        out_shape=jax.ShapeDtypeStruct((M, D_MODEL), jnp.float32),
    )(te.reshape(128), x_disp, wfc, c_fc_b.reshape(N_EXPERT, 1, D_FF),
      wproj, c_proj_b.reshape(N_EXPERT, 1, D_MODEL), w_sorted)

    final = _get_combine()(y, posk_flat)
    return (final.reshape(b, s, d), logits)


# speedup vs baseline: 1.8143x; 1.1415x over previous
import functools

import jax
import jax.numpy as jnp
from jax import lax
from jax.experimental import pallas as pl
from jax.experimental.pallas import tpu as pltpu
from jax.experimental.pallas import tpu_sc as plsc

S = 2048
D_MODEL = 1024
D_FF = 4096
N_EXPERT = 8
TOP_K = 2
ROW_TILE = 256
BLK = 256
M = S * TOP_K + N_EXPERT * ROW_TILE
NT = M // ROW_TILE


def _gelu_new(x):
    return 0.5 * x * (1.0 + jnp.tanh(jnp.sqrt(2.0 / jnp.pi) * (x + 0.044715 * jnp.power(x, 3.0))))


def _router_body(x_ref, wg_ref, logits_ref, posk_ref, wpw_ref, te_ref,
                 oh0_s, oh1_s, r0_s, r1_s):
    x = x_ref[...]
    logits = lax.dot_general(x, wg_ref[...], (((1,), (0,)), ((), ())),
                             preferred_element_type=jnp.float32)
    logits_ref[...] = logits
    m = jnp.max(logits, axis=1, keepdims=True)
    ex = jnp.exp(logits - m)
    probs = ex / jnp.sum(ex, axis=1, keepdims=True)
    iota8 = lax.broadcasted_iota(jnp.int32, (S, N_EXPERT), 1)
    m1 = jnp.max(probs, axis=1, keepdims=True)
    e1 = jnp.min(jnp.where(probs == m1, iota8, N_EXPERT), axis=1, keepdims=True)
    masked = jnp.where(iota8 == e1, -jnp.inf, probs)
    m2 = jnp.max(masked, axis=1, keepdims=True)
    e2 = jnp.min(jnp.where(masked == m2, iota8, N_EXPERT), axis=1, keepdims=True)
    denom = m1 + m2
    w1 = m1 / denom
    w2 = m2 / denom

    oh0_s[...] = (iota8 == e1).astype(jnp.float32)
    oh1_s[...] = (iota8 == e2).astype(jnp.float32)

    rb = (lax.broadcasted_iota(jnp.int32, (BLK, BLK), 0)
          > lax.broadcasted_iota(jnp.int32, (BLK, BLK), 1)).astype(jnp.float32)

    def step(i, carry):
        c0, c1 = carry
        b0 = oh0_s[pl.ds(i * BLK, BLK), :]
        b1 = oh1_s[pl.ds(i * BLK, BLK), :]
        r0_s[pl.ds(i * BLK, BLK), :] = lax.dot_general(
            rb, b0, (((1,), (0,)), ((), ())), preferred_element_type=jnp.float32) + c0
        r1_s[pl.ds(i * BLK, BLK), :] = lax.dot_general(
            rb, b1, (((1,), (0,)), ((), ())), preferred_element_type=jnp.float32) + c1
        return (c0 + jnp.sum(b0, axis=0, keepdims=True),
                c1 + jnp.sum(b1, axis=0, keepdims=True))

    c0, c1 = lax.fori_loop(0, S // BLK, step,
                           (jnp.zeros((1, N_EXPERT), jnp.float32),
                            jnp.zeros((1, N_EXPERT), jnp.float32)))
    counts = c0 + c1
    pc = jnp.ceil(counts / ROW_TILE) * ROW_TILE
    tri8 = (lax.broadcasted_iota(jnp.int32, (N_EXPERT, N_EXPERT), 0)
            < lax.broadcasted_iota(jnp.int32, (N_EXPERT, N_EXPERT), 1)).astype(jnp.float32)
    po = lax.dot_general(pc, tri8, (((1,), (0,)), ((), ())),
                         preferred_element_type=jnp.float32)

    oh0 = oh0_s[...]
    oh1 = oh1_s[...]

    def sel(mat, oh):
        return jnp.sum(mat * oh, axis=1, keepdims=True)

    pos0 = sel(po, oh0) + sel(r0_s[...], oh0)
    pos1 = sel(po, oh1) + sel(c0, oh1) + sel(r1_s[...], oh1)
    posk = (jnp.where(iota8 == 0, pos0, 0.0)
            + jnp.where(iota8 == 1, pos1, 0.0))
    posk_ref[...] = jnp.transpose(posk).astype(jnp.int32)
    wpw_ref[pl.ds(0, S), :] = jnp.broadcast_to(w1, (S, 128))
    wpw_ref[pl.ds(S, S), :] = jnp.broadcast_to(w2, (S, 128))

    iota128 = lax.broadcasted_iota(jnp.int32, (1, 128), 1)
    ti = iota128.astype(jnp.float32) * ROW_TILE
    iota_e = lax.broadcasted_iota(jnp.int32, (1, N_EXPERT), 1)
    te = jnp.zeros((1, 128), jnp.float32)
    last_e = jnp.float32(0.0)
    for e in range(N_EXPERT):
        oh_e = (iota_e == e).astype(jnp.float32)
        po_e = jnp.sum(po * oh_e)
        pc_e = jnp.sum(pc * oh_e)
        te += (ti >= po_e).astype(jnp.float32)
        last_e = jnp.maximum(last_e, jnp.where(pc_e > 0, jnp.float32(e), 0.0))
    te = jnp.minimum(te - 1.0, last_e)
    te = jnp.maximum(te, 0.0)
    n_used = jnp.sum(pc) / ROW_TILE
    te = jnp.where(iota128 == 127, n_used, te)
    te_ref[...] = te.astype(jnp.int32)


@functools.lru_cache(maxsize=None)
def _get_dispatch():
    mesh = plsc.VectorSubcoreMesh(core_axis_name="c", subcore_axis_name="s")

    @functools.partial(
        pl.kernel, mesh=mesh,
        out_type=(
            jax.ShapeDtypeStruct((M, D_MODEL), jnp.float32),
            jax.ShapeDtypeStruct((M, 128), jnp.float32),
        ),
        scratch_types=[
            pltpu.VMEM((32,), jnp.int32),
            pltpu.VMEM((32,), jnp.int32),
            pltpu.VMEM((32, D_MODEL), jnp.float32),
            pltpu.VMEM((32, D_MODEL), jnp.float32),
            pltpu.VMEM((32, 128), jnp.float32),
            pltpu.VMEM((32, 128), jnp.float32),
            pltpu.SemaphoreType.DMA,
            pltpu.SemaphoreType.DMA,
        ],
    )
    def _dispatch(hs_hbm, posk_hbm, wpw_hbm, xd_hbm, ws_hbm,
                  idx0_v, idx1_v, rows0_v, rows1_v, w0_v, w1_v, sem_in, sem_out):
        wid = lax.axis_index("s") * 2 + lax.axis_index("c")
        k = wid // 16
        t0 = (wid % 16) * 128
        CH = 32
        bufs = ((idx0_v, rows0_v, w0_v), (idx1_v, rows1_v, w1_v))

        def stage(c, bi):
            i_v, r_v, w_v = bufs[bi]
            tc = t0 + c * CH
            pltpu.sync_copy(posk_hbm.at[pl.ds(k * S + tc, CH)], i_v)
            return (pltpu.async_copy(hs_hbm.at[pl.ds(tc, CH)], r_v, sem_in),
                    pltpu.async_copy(wpw_hbm.at[pl.ds(k * S + tc, CH)], w_v, sem_in))

        def scatter(bi):
            i_v, r_v, w_v = bufs[bi]
            return (pltpu.async_copy(r_v, xd_hbm.at[i_v], sem_out),
                    pltpu.async_copy(w_v, ws_hbm.at[i_v], sem_out))

        pend_sc = None
        st = stage(0, 0)
        for c in range(4):
            bi = c % 2
            if pend_sc is not None:
                pend_sc[0].wait()
                pend_sc[1].wait()
            stn = stage(c + 1, 1 - bi) if c < 3 else None
            st[0].wait()
            st[1].wait()
            pend_sc = scatter(bi)
            st = stn
        pend_sc[0].wait()
        pend_sc[1].wait()

    return _dispatch


def _cast_body(wfc_ref, ofc_ref):
    ofc_ref[...] = wfc_ref[...].astype(jnp.bfloat16)


def _cast_weights(c_fc_w):
    return pl.pallas_call(
        _cast_body,
        grid=(N_EXPERT, 4),
        in_specs=[
            pl.BlockSpec((1, D_MODEL // 4, D_FF), lambda e, c: (e, c, 0)),
        ],
        out_specs=pl.BlockSpec((1, D_MODEL // 4, D_FF), lambda e, c: (e, c, 0)),
        out_shape=jax.ShapeDtypeStruct((N_EXPERT, D_MODEL, D_FF), jnp.bfloat16),
    )(c_fc_w)


def _ffn_body(te_ref, x_ref, wfc_ref, bfc_ref, wproj_ref, bproj_ref, ws_ref, y_ref):
    i = pl.program_id(0)

    @pl.when(i < te_ref[127])
    def _():
        x = x_ref[...].astype(jnp.bfloat16)
        h = lax.dot_general(x, wfc_ref[0], (((1,), (0,)), ((), ())),
                            preferred_element_type=jnp.float32)
        h = _gelu_new(h + bfc_ref[0])
        y = lax.dot_general(h, wproj_ref[0], (((1,), (0,)), ((), ())),
                            preferred_element_type=jnp.float32)
        w_col = ws_ref[...][:, 0:1]
        y_ref[...] = (y + bproj_ref[0]) * w_col


@functools.lru_cache(maxsize=None)
def _get_combine():
    mesh = plsc.VectorSubcoreMesh(core_axis_name="c", subcore_axis_name="s")

    @functools.partial(
        pl.kernel, mesh=mesh,
        out_type=jax.ShapeDtypeStruct((S, D_MODEL), jnp.float32),
        scratch_types=[
            pltpu.VMEM((16,), jnp.int32),
            pltpu.VMEM((16,), jnp.int32),
            pltpu.VMEM((16,), jnp.int32),
            pltpu.VMEM((16,), jnp.int32),
            pltpu.VMEM((16, D_MODEL), jnp.float32),
            pltpu.VMEM((16, D_MODEL), jnp.float32),
            pltpu.VMEM((16, D_MODEL), jnp.float32),
            pltpu.VMEM((16, D_MODEL), jnp.float32),
            pltpu.VMEM((16, D_MODEL), jnp.float32),
            pltpu.VMEM((16, D_MODEL), jnp.float32),
            pltpu.SemaphoreType.DMA,
        ],
    )
    def _combine(y_hbm, posk_hbm, out_hbm,
                 idx0a_v, idx1a_v, idx0b_v, idx1b_v,
                 y0a_v, y1a_v, y0b_v, y1b_v, oa_v, ob_v, sem):
        wid = lax.axis_index("s") * 2 + lax.axis_index("c")
        t0 = wid * 64
        CH = 16
        bufs = ((idx0a_v, idx1a_v, y0a_v, y1a_v, oa_v),
                (idx0b_v, idx1b_v, y0b_v, y1b_v, ob_v))

        def fire(c, bi):
            i0, i1, yy0, yy1, _ = bufs[bi]
            tc = t0 + c * CH
            pltpu.sync_copy(posk_hbm.at[pl.ds(tc, CH)], i0)
            pltpu.sync_copy(posk_hbm.at[pl.ds(S + tc, CH)], i1)
            return (pltpu.async_copy(y_hbm.at[i0], yy0, sem),
                    pltpu.async_copy(y_hbm.at[i1], yy1, sem))

        pend = fire(0, 0)
        for c in range(4):
            bi = c % 2
            nxt = fire(c + 1, 1 - bi) if c < 3 else None
            pend[0].wait()
            pend[1].wait()
            _, _, yy0, yy1, oo = bufs[bi]

            def tok(i, carry, yy0=yy0, yy1=yy1, oo=oo):
                for cc in range(D_MODEL // 16):
                    oo[i, pl.ds(cc * 16, 16)] = (yy0[i, pl.ds(cc * 16, 16)]
                                                 + yy1[i, pl.ds(cc * 16, 16)])
                return carry

            lax.fori_loop(0, CH, tok, 0)
            pltpu.sync_copy(oo, out_hbm.at[pl.ds(t0 + c * CH, CH)])
            pend = nxt

    return _combine


def kernel(hidden_states, W_g, c_fc_w, c_fc_b, c_proj_w, c_proj_b):
    b, s, d = hidden_states.shape
    hs = hidden_states.reshape(s, d)

    logits, posk, wpw, te = pl.pallas_call(
        _router_body,
        out_shape=(
            jax.ShapeDtypeStruct((S, N_EXPERT), jnp.float32),
            jax.ShapeDtypeStruct((N_EXPERT, S), jnp.int32),
            jax.ShapeDtypeStruct((TOP_K * S, 128), jnp.float32),
            jax.ShapeDtypeStruct((1, 128), jnp.int32),
        ),
        scratch_shapes=[pltpu.VMEM((S, N_EXPERT), jnp.float32)] * 4,
    )(hs, W_g)

    posk_flat = posk[:TOP_K].reshape(TOP_K * S)
    x_disp, w_sorted = _get_dispatch()(hs, posk_flat, wpw)

    wfc = _cast_weights(c_fc_w)

    y = pl.pallas_call(
        _ffn_body,
        grid_spec=pltpu.PrefetchScalarGridSpec(
            num_scalar_prefetch=1,
            grid=(NT,),
            in_specs=[
                pl.BlockSpec((ROW_TILE, D_MODEL), lambda i, te: (i, 0)),
                pl.BlockSpec((1, D_MODEL, D_FF), lambda i, te: (te[i], 0, 0)),
                pl.BlockSpec((1, 1, D_FF), lambda i, te: (te[i], 0, 0)),
                pl.BlockSpec((1, D_FF, D_MODEL), lambda i, te: (te[i], 0, 0)),
                pl.BlockSpec((1, 1, D_MODEL), lambda i, te: (te[i], 0, 0)),
                pl.BlockSpec((ROW_TILE, 128), lambda i, te: (i, 0)),
            ],
            out_specs=pl.BlockSpec((ROW_TILE, D_MODEL), lambda i, te: (i, 0)),
        ),
        out_shape=jax.ShapeDtypeStruct((M, D_MODEL), jnp.float32),
    )(te.reshape(128), x_disp, wfc, c_fc_b.reshape(N_EXPERT, 1, D_FF),
      c_proj_w, c_proj_b.reshape(N_EXPERT, 1, D_MODEL), w_sorted)

    final = _get_combine()(y, posk_flat)
    return (final.reshape(b, s, d), logits)
